# baseline (speedup 1.0000x reference)
import functools

import jax
import jax.numpy as jnp
from jax import lax
from jax.experimental import pallas as pl
from jax.experimental.pallas import tpu as pltpu
from jax.experimental.pallas import tpu_sc as plsc

_LANES = 16
_BB = 128
_UNITS_PER_GROUP = 200


@functools.lru_cache(maxsize=None)
def _build_lookup(n_t: int, n_b: int, v: int, d: int):
    info = plsc.get_sparse_core_info()
    nc, ns = info.num_cores, info.num_subcores
    nw = nc * ns
    n_units = n_t * (n_b // _BB)
    assert n_units % (nw * _UNITS_PER_GROUP) == 0
    units_per_w = n_units // nw
    groups_per_w = units_per_w // _UNITS_PER_GROUP
    btiles = n_b // _BB
    dtiles = d // 8
    slab = 8 * _BB
    mesh = plsc.VectorSubcoreMesh(core_axis_name="c", subcore_axis_name="s")

    @functools.partial(
        pl.kernel,
        mesh=mesh,
        out_type=jax.ShapeDtypeStruct((n_t, dtiles, btiles, slab), jnp.float32),
        scratch_types=[
            pltpu.VMEM((v * d,), jnp.float32),
            pltpu.VMEM((_UNITS_PER_GROUP * _BB,), jnp.int32),
            pltpu.VMEM((2, dtiles * slab), jnp.float32),
            pltpu.SemaphoreType.DMA,
            pltpu.SemaphoreType.DMA,
        ],
        compiler_params=pltpu.CompilerParams(
            use_tc_tiling_on_sc=False, needs_layout_passes=False),
    )
    def lookup(ids_hbm, table_hbm, out_hbm, wt_v, idx_v, blk_v, sem0, sem1):
        wid = lax.axis_index("s") * nc + lax.axis_index("c")
        pltpu.sync_copy(table_hbm, wt_v)
        sems = (sem0, sem1)

        def drain(tb):
            for _ in range(dtiles):
                pltpu.make_async_copy(
                    out_hbm.at[0].at[0].at[0],
                    blk_v.at[tb].at[pl.ds(0, slab)],
                    sems[tb],
                ).wait()

        def unit(u, jl, tb):
            t = u // btiles
            btile = u - t * btiles
            n_sub = (_BB // _LANES) * (d // _LANES)

            @plsc.parallel_loop(0, n_sub, 1, unroll=4)
            def _(i):
                k = i // (d // _LANES)
                di = i % (d // _LANES)
                ids16 = idx_v[pl.ds(jl * _BB + k * _LANES, _LANES)]
                off = di * _LANES
                soff = off * _BB + k * _LANES
                for dd in range(_LANES):
                    val = plsc.load_gather(wt_v, [ids16 + (off + dd) * v])
                    blk_v.at[tb][pl.ds(soff + dd * _BB, _LANES)] = val
            for dt in range(dtiles):
                pltpu.async_copy(
                    blk_v.at[tb].at[pl.ds(dt * slab, slab)],
                    out_hbm.at[t].at[dt].at[btile],
                    sems[tb],
                )

        def group_body(g, carry):
            u0 = wid * units_per_w + g * _UNITS_PER_GROUP
            pltpu.sync_copy(
                ids_hbm.at[pl.ds(u0 * _BB, _UNITS_PER_GROUP * _BB)], idx_v)

            @pl.when(g == 0)
            def _():
                unit(u0, 0, 0)
                unit(u0 + 1, 1, 1)

            @pl.when(g > 0)
            def _():
                for joff in (0, 1):
                    drain(joff)
                    unit(u0 + joff, joff, joff)

            def pair_body(p, carry2):
                for joff in (0, 1):
                    jl = 2 * p + joff
                    drain(joff)
                    unit(u0 + jl, jl, joff)
                return carry2

            lax.fori_loop(1, _UNITS_PER_GROUP // 2, pair_body, 0)
            return carry

        lax.fori_loop(0, groups_per_w, group_body, 0)
        drain(0)
        drain(1)

    return lookup


def kernel(input_ids, W):
    bsz, seq = input_ids.shape
    v, d = W.shape
    ids_t = input_ids.astype(jnp.int32).T.reshape(-1)
    wt = W.T.reshape(-1)
    out5 = _build_lookup(seq, bsz, v, d)(ids_t, wt)
    out6 = out5.reshape(seq, d // 8, bsz // _BB, 8, _BB)
    out = out6.transpose(2, 4, 0, 1, 3).reshape(bsz, seq, d)
    return out


# speedup vs baseline: 1.0076x; 1.0076x over previous
import functools

import jax
import jax.numpy as jnp
from jax import lax
from jax.experimental import pallas as pl
from jax.experimental.pallas import tpu as pltpu
from jax.experimental.pallas import tpu_sc as plsc

_LANES = 16
_BB = 128
_UNITS_PER_GROUP = 200


@functools.lru_cache(maxsize=None)
def _build_lookup(n_t: int, n_b: int, v: int, d: int):
    info = plsc.get_sparse_core_info()
    nc, ns = info.num_cores, info.num_subcores
    nw = nc * ns
    n_units = n_t * (n_b // _BB)
    assert n_units % (nw * _UNITS_PER_GROUP) == 0
    units_per_w = n_units // nw
    groups_per_w = units_per_w // _UNITS_PER_GROUP
    btiles = n_b // _BB
    dtiles = d // 8
    slab = 8 * _BB
    mesh = plsc.VectorSubcoreMesh(core_axis_name="c", subcore_axis_name="s")

    @functools.partial(
        pl.kernel,
        mesh=mesh,
        out_type=jax.ShapeDtypeStruct((n_t, dtiles, btiles, slab), jnp.float32),
        scratch_types=[
            pltpu.VMEM((v * d,), jnp.float32),
            pltpu.VMEM((_UNITS_PER_GROUP * _BB,), jnp.int32),
            pltpu.VMEM((2, dtiles * slab), jnp.float32),
            pltpu.SemaphoreType.DMA,
            pltpu.SemaphoreType.DMA,
        ],
        compiler_params=pltpu.CompilerParams(
            use_tc_tiling_on_sc=False, needs_layout_passes=False),
    )
    def lookup(ids_hbm, table_hbm, out_hbm, wt_v, idx_v, blk_v, sem0, sem1):
        wid = lax.axis_index("s") * nc + lax.axis_index("c")
        pltpu.sync_copy(table_hbm, wt_v)
        sems = (sem0, sem1)

        def drain(tb):
            for _ in range(dtiles):
                pltpu.make_async_copy(
                    out_hbm.at[0].at[0].at[0],
                    blk_v.at[tb].at[pl.ds(0, slab)],
                    sems[tb],
                ).wait()

        def unit(u, jl, tb):
            t = u // btiles
            btile = u - t * btiles
            n_sub = (_BB // _LANES) * (d // _LANES)

            @plsc.parallel_loop(0, n_sub, 1, unroll=4)
            def _(i):
                k = i // (d // _LANES)
                di = i % (d // _LANES)
                ids16 = idx_v[pl.ds(jl * _BB + k * _LANES, _LANES)]
                off = di * _LANES
                soff = off * _BB + k * _LANES
                for dd in range(_LANES):
                    val = plsc.load_gather(wt_v, [ids16 + (off + dd) * v])
                    blk_v.at[tb][pl.ds(soff + dd * _BB, _LANES)] = val
            for dt in range(dtiles):
                pltpu.async_copy(
                    blk_v.at[tb].at[pl.ds(dt * slab, slab)],
                    out_hbm.at[t].at[dt].at[btile],
                    sems[tb],
                )

        def group_body(g, carry):
            u0 = wid * units_per_w + g * _UNITS_PER_GROUP
            pltpu.sync_copy(
                ids_hbm.at[pl.ds(u0 * _BB, _UNITS_PER_GROUP * _BB)], idx_v)

            def pair_body(p, carry2):
                for joff in (0, 1):
                    jl = 2 * p + joff

                    @pl.when(jnp.logical_or(g > 0, p > 0))
                    def _():
                        drain(joff)

                    unit(u0 + jl, jl, joff)
                return carry2

            lax.fori_loop(0, _UNITS_PER_GROUP // 2, pair_body, 0)
            return carry

        lax.fori_loop(0, groups_per_w, group_body, 0)
        drain(0)
        drain(1)

    return lookup


def kernel(input_ids, W):
    bsz, seq = input_ids.shape
    v, d = W.shape
    ids_t = input_ids.astype(jnp.int32).T.reshape(-1)
    wt = W.T.reshape(-1)
    out5 = _build_lookup(seq, bsz, v, d)(ids_t, wt)
    out6 = out5.reshape(seq, d // 8, bsz // _BB, 8, _BB)
    out = out6.transpose(2, 4, 0, 1, 3).reshape(bsz, seq, d)
    return out
